# Initial kernel scaffold; baseline (speedup 1.0000x reference)
import jax
import jax.numpy as jnp
from jax.experimental import pallas as pl


def kernel(query, key, value, Wg, Wq, Wk, Wv, Wo, rel_pos_emb):
    raise NotImplementedError("write your pallas kernel here")


# speedup vs baseline: 4089.3269x; 4089.3269x over previous
import jax
import jax.numpy as jnp
from jax.experimental import pallas as pl

EMBED_DIM = 1024
NUM_EXPERT = 16
TOP_K = 2
EXPERT_DIM = 256
HEAD_DIM = 64
NUM_HEADS = EXPERT_DIM // HEAD_DIM
MAX_POS = 64
SCALING = HEAD_DIM ** (-0.25)


def _gate_kernel(x_ref, wg_ref, idx_ref, gate_ref):
    logits = x_ref[...] @ wg_ref[...]
    e_iota = jax.lax.broadcasted_iota(jnp.int32, logits.shape, 1)
    m1 = jnp.max(logits, axis=1, keepdims=True)
    i1 = jnp.min(jnp.where(logits == m1, e_iota, NUM_EXPERT), axis=1,
                 keepdims=True)
    masked = jnp.where(e_iota == i1, -jnp.inf, logits)
    m2 = jnp.max(masked, axis=1, keepdims=True)
    i2 = jnp.min(jnp.where(masked == m2, e_iota, NUM_EXPERT), axis=1,
                 keepdims=True)
    g1 = jax.nn.sigmoid(m1 - m2)
    idx_ref[...] = jnp.concatenate([i1, i2], axis=1)
    gate_ref[...] = jnp.concatenate([g1, 1.0 - g1], axis=1)


def _qproj_kernel(x_ref, wq_ref, idx_ref, q_ref):
    e = pl.program_id(1)

    @pl.when(e == 0)
    def _():
        q_ref[...] = jnp.zeros_like(q_ref)

    p = x_ref[...] @ wq_ref[0]
    idx = idx_ref[...]
    for k in range(TOP_K):
        sel = idx[:, k:k + 1] == e
        q_ref[:, k * EXPERT_DIM:(k + 1) * EXPERT_DIM] += jnp.where(sel, p, 0.0)


def _kv_kernel(xk_ref, xv_ref, wk_ref, wv_ref, k_ref, v_ref):
    k_ref[...] = xk_ref[...] @ wk_ref[...]
    v_ref[...] = xv_ref[...] @ wv_ref[...]


def _attn_kernel(q_ref, k_ref, v_ref, e_ref, y_ref, *, block_t):
    t = pl.program_id(1)
    qs = q_ref[0] * SCALING
    scores = jax.lax.dot_general(
        qs, k_ref[0], (((1,), (1,)), ((), ()))) * SCALING
    rlog = qs @ e_ref[0]
    i0 = t * block_t
    ii = jax.lax.broadcasted_iota(jnp.int32, scores.shape, 0) + i0
    jj = jax.lax.broadcasted_iota(jnp.int32, scores.shape, 1)
    d = jnp.clip(jj - ii, 1 - MAX_POS, MAX_POS - 1) + MAX_POS
    rel = jnp.take_along_axis(rlog, d, axis=1)
    scores = scores + rel
    m = jnp.max(scores, axis=1, keepdims=True)
    p = jnp.exp(scores - m)
    l = jnp.sum(p, axis=1, keepdims=True)
    y_ref[0] = (p @ v_ref[0]) / l


def _oproj_kernel(y_ref, idx_ref, gate_ref, wo_ref, o_ref):
    e = pl.program_id(1)

    @pl.when(e == 0)
    def _():
        o_ref[...] = jnp.zeros_like(o_ref)

    idx = idx_ref[...]
    g = gate_ref[...]
    z = None
    for k in range(TOP_K):
        w = jnp.where(idx[:, k:k + 1] == e, g[:, k:k + 1], 0.0)
        zk = y_ref[:, k * EXPERT_DIM:(k + 1) * EXPERT_DIM] * w
        z = zk if z is None else z + zk
    o_ref[...] += z @ wo_ref[0]


def kernel(query, key, value, Wg, Wq, Wk, Wv, Wo, rel_pos_emb):
    T, B, D = query.shape
    S = key.shape[0]
    n = T * B
    x = query.reshape(n, D)
    xk = key.reshape(S * B, D)
    xv = value.reshape(S * B, D)

    bT = 512
    f32 = jnp.float32

    idx, gates = pl.pallas_call(
        _gate_kernel,
        grid=(n // bT,),
        in_specs=[
            pl.BlockSpec((bT, D), lambda i: (i, 0)),
            pl.BlockSpec((D, NUM_EXPERT), lambda i: (0, 0)),
        ],
        out_specs=[
            pl.BlockSpec((bT, TOP_K), lambda i: (i, 0)),
            pl.BlockSpec((bT, TOP_K), lambda i: (i, 0)),
        ],
        out_shape=[
            jax.ShapeDtypeStruct((n, TOP_K), jnp.int32),
            jax.ShapeDtypeStruct((n, TOP_K), f32),
        ],
    )(x, Wg)

    q = pl.pallas_call(
        _qproj_kernel,
        grid=(n // bT, NUM_EXPERT),
        in_specs=[
            pl.BlockSpec((bT, D), lambda i, e: (i, 0)),
            pl.BlockSpec((1, D, EXPERT_DIM), lambda i, e: (e, 0, 0)),
            pl.BlockSpec((bT, TOP_K), lambda i, e: (i, 0)),
        ],
        out_specs=pl.BlockSpec((bT, TOP_K * EXPERT_DIM), lambda i, e: (i, 0)),
        out_shape=jax.ShapeDtypeStruct((n, TOP_K * EXPERT_DIM), f32),
    )(x, Wq, idx)

    kp, vp = pl.pallas_call(
        _kv_kernel,
        grid=(S * B // bT,),
        in_specs=[
            pl.BlockSpec((bT, D), lambda i: (i, 0)),
            pl.BlockSpec((bT, D), lambda i: (i, 0)),
            pl.BlockSpec((D, EXPERT_DIM), lambda i: (0, 0)),
            pl.BlockSpec((D, EXPERT_DIM), lambda i: (0, 0)),
        ],
        out_specs=[
            pl.BlockSpec((bT, EXPERT_DIM), lambda i: (i, 0)),
            pl.BlockSpec((bT, EXPERT_DIM), lambda i: (i, 0)),
        ],
        out_shape=[
            jax.ShapeDtypeStruct((S * B, EXPERT_DIM), f32),
            jax.ShapeDtypeStruct((S * B, EXPERT_DIM), f32),
        ],
    )(xk, xv, Wk, Wv)

    bA = 256
    import functools
    KH = TOP_K * NUM_HEADS
    q3 = q.reshape(n, KH, HEAD_DIM).transpose(1, 0, 2)
    rpe = rel_pos_emb[:, :, :2 * MAX_POS]
    k3 = kp.reshape(S * B, NUM_HEADS, HEAD_DIM).transpose(1, 0, 2)
    v3 = vp.reshape(S * B, NUM_HEADS, HEAD_DIM).transpose(1, 0, 2)
    y3 = pl.pallas_call(
        functools.partial(_attn_kernel, block_t=bA),
        grid=(KH, T // bA),
        in_specs=[
            pl.BlockSpec((1, bA, HEAD_DIM), lambda kh, t: (kh, t, 0)),
            pl.BlockSpec((1, S, HEAD_DIM), lambda kh, t: (kh % NUM_HEADS, 0, 0)),
            pl.BlockSpec((1, S, HEAD_DIM), lambda kh, t: (kh % NUM_HEADS, 0, 0)),
            pl.BlockSpec((1, HEAD_DIM, 2 * MAX_POS),
                         lambda kh, t: (kh % NUM_HEADS, 0, 0)),
        ],
        out_specs=pl.BlockSpec((1, bA, HEAD_DIM), lambda kh, t: (kh, t, 0)),
        out_shape=jax.ShapeDtypeStruct((KH, n, HEAD_DIM), f32),
    )(q3, k3, v3, rpe)
    y = y3.transpose(1, 0, 2).reshape(n, TOP_K * EXPERT_DIM)

    out = pl.pallas_call(
        _oproj_kernel,
        grid=(n // bT, NUM_EXPERT),
        in_specs=[
            pl.BlockSpec((bT, TOP_K * EXPERT_DIM), lambda i, e: (i, 0)),
            pl.BlockSpec((bT, TOP_K), lambda i, e: (i, 0)),
            pl.BlockSpec((bT, TOP_K), lambda i, e: (i, 0)),
            pl.BlockSpec((1, EXPERT_DIM, D), lambda i, e: (e, 0, 0)),
        ],
        out_specs=pl.BlockSpec((bT, D), lambda i, e: (i, 0)),
        out_shape=jax.ShapeDtypeStruct((n, D), f32),
    )(y, idx, gates, Wo)

    return out.reshape(T, B, D)


# speedup vs baseline: 4383.2657x; 1.0719x over previous
import functools

import jax
import jax.numpy as jnp
from jax.experimental import pallas as pl

EMBED_DIM = 1024
NUM_EXPERT = 16
TOP_K = 2
EXPERT_DIM = 256
HEAD_DIM = 64
NUM_HEADS = EXPERT_DIM // HEAD_DIM
MAX_POS = 64
SCALING = HEAD_DIM ** (-0.25)


def _gate_kernel(x_ref, wg_ref, idx_ref, gate_ref):
    logits = x_ref[...] @ wg_ref[...]
    e_iota = jax.lax.broadcasted_iota(jnp.int32, logits.shape, 1)
    m1 = jnp.max(logits, axis=1, keepdims=True)
    i1 = jnp.min(jnp.where(logits == m1, e_iota, NUM_EXPERT), axis=1,
                 keepdims=True)
    masked = jnp.where(e_iota == i1, -jnp.inf, logits)
    m2 = jnp.max(masked, axis=1, keepdims=True)
    i2 = jnp.min(jnp.where(masked == m2, e_iota, NUM_EXPERT), axis=1,
                 keepdims=True)
    g1 = jax.nn.sigmoid(m1 - m2)
    idx_ref[...] = jnp.concatenate([i1, i2], axis=1)
    gate_ref[...] = jnp.concatenate([g1, 1.0 - g1], axis=1)


def _qproj_kernel(x_ref, wq_ref, idx_ref, q_ref):
    e = pl.program_id(1)

    @pl.when(e == 0)
    def _():
        q_ref[...] = jnp.zeros_like(q_ref)

    p = (x_ref[...] @ wq_ref[0]) * SCALING
    idx = idx_ref[...]
    for k in range(TOP_K):
        ps = jnp.where(idx[:, k:k + 1] == e, p, 0.0)
        for h in range(NUM_HEADS):
            q_ref[k * NUM_HEADS + h] += ps[:, h * HEAD_DIM:(h + 1) * HEAD_DIM]


def _kv_kernel(xk_ref, xv_ref, wk_ref, wv_ref, k_ref, v_ref):
    kp = (xk_ref[...] @ wk_ref[...]) * SCALING
    vp = xv_ref[...] @ wv_ref[...]
    for h in range(NUM_HEADS):
        k_ref[h] = kp[:, h * HEAD_DIM:(h + 1) * HEAD_DIM]
        v_ref[h] = vp[:, h * HEAD_DIM:(h + 1) * HEAD_DIM]


def _attn_kernel(q_ref, k_ref, v_ref, e_ref, y_ref, *, block_t):
    t = pl.program_id(1)
    qs = q_ref[0]
    scores = jax.lax.dot_general(
        qs, k_ref[0], (((1,), (1,)), ((), ())))
    rlog = qs @ e_ref[0]
    i0 = t * block_t
    ii = jax.lax.broadcasted_iota(jnp.int32, scores.shape, 0) + i0
    jj = jax.lax.broadcasted_iota(jnp.int32, scores.shape, 1)
    d = jnp.clip(jj - ii, 1 - MAX_POS, MAX_POS - 1) + MAX_POS
    scores = scores + jnp.take_along_axis(rlog, d, axis=1)
    m = jnp.max(scores, axis=1, keepdims=True)
    p = jnp.exp(scores - m)
    l = jnp.sum(p, axis=1, keepdims=True)
    y_ref[0] = (p @ v_ref[0]) / l


def _oproj_kernel(y_ref, idx_ref, gate_ref, wo_ref, o_ref):
    e = pl.program_id(1)

    @pl.when(e == 0)
    def _():
        o_ref[...] = jnp.zeros_like(o_ref)

    idx = idx_ref[...]
    g = gate_ref[...]
    z = None
    for k in range(TOP_K):
        w = jnp.where(idx[:, k:k + 1] == e, g[:, k:k + 1], 0.0)
        yk = jnp.concatenate(
            [y_ref[k * NUM_HEADS + h] for h in range(NUM_HEADS)], axis=1)
        zk = yk * w
        z = zk if z is None else z + zk
    o_ref[...] += z @ wo_ref[0]


def kernel(query, key, value, Wg, Wq, Wk, Wv, Wo, rel_pos_emb):
    T, B, D = query.shape
    S = key.shape[0]
    n = T * B
    x = query.reshape(n, D)
    xk = key.reshape(S * B, D)
    xv = value.reshape(S * B, D)
    KH = TOP_K * NUM_HEADS
    f32 = jnp.float32

    bT = 512
    idx, gates = pl.pallas_call(
        _gate_kernel,
        grid=(n // bT,),
        in_specs=[
            pl.BlockSpec((bT, D), lambda i: (i, 0)),
            pl.BlockSpec((D, NUM_EXPERT), lambda i: (0, 0)),
        ],
        out_specs=[
            pl.BlockSpec((bT, TOP_K), lambda i: (i, 0)),
            pl.BlockSpec((bT, TOP_K), lambda i: (i, 0)),
        ],
        out_shape=[
            jax.ShapeDtypeStruct((n, TOP_K), jnp.int32),
            jax.ShapeDtypeStruct((n, TOP_K), f32),
        ],
    )(x, Wg)

    q3 = pl.pallas_call(
        _qproj_kernel,
        grid=(n // bT, NUM_EXPERT),
        in_specs=[
            pl.BlockSpec((bT, D), lambda i, e: (i, 0)),
            pl.BlockSpec((1, D, EXPERT_DIM), lambda i, e: (e, 0, 0)),
            pl.BlockSpec((bT, TOP_K), lambda i, e: (i, 0)),
        ],
        out_specs=pl.BlockSpec((KH, bT, HEAD_DIM), lambda i, e: (0, i, 0)),
        out_shape=jax.ShapeDtypeStruct((KH, n, HEAD_DIM), f32),
    )(x, Wq, idx)

    k3, v3 = pl.pallas_call(
        _kv_kernel,
        grid=(S * B // bT,),
        in_specs=[
            pl.BlockSpec((bT, D), lambda i: (i, 0)),
            pl.BlockSpec((bT, D), lambda i: (i, 0)),
            pl.BlockSpec((D, EXPERT_DIM), lambda i: (0, 0)),
            pl.BlockSpec((D, EXPERT_DIM), lambda i: (0, 0)),
        ],
        out_specs=[
            pl.BlockSpec((NUM_HEADS, bT, HEAD_DIM), lambda i: (0, i, 0)),
            pl.BlockSpec((NUM_HEADS, bT, HEAD_DIM), lambda i: (0, i, 0)),
        ],
        out_shape=[
            jax.ShapeDtypeStruct((NUM_HEADS, S * B, HEAD_DIM), f32),
            jax.ShapeDtypeStruct((NUM_HEADS, S * B, HEAD_DIM), f32),
        ],
    )(xk, xv, Wk, Wv)

    rpe = rel_pos_emb[:, :, :2 * MAX_POS]

    bA = 256
    y3 = pl.pallas_call(
        functools.partial(_attn_kernel, block_t=bA),
        grid=(KH, T // bA),
        in_specs=[
            pl.BlockSpec((1, bA, HEAD_DIM), lambda kh, t: (kh, t, 0)),
            pl.BlockSpec((1, S, HEAD_DIM), lambda kh, t: (kh % NUM_HEADS, 0, 0)),
            pl.BlockSpec((1, S, HEAD_DIM), lambda kh, t: (kh % NUM_HEADS, 0, 0)),
            pl.BlockSpec((1, HEAD_DIM, 2 * MAX_POS),
                         lambda kh, t: (kh % NUM_HEADS, 0, 0)),
        ],
        out_specs=pl.BlockSpec((1, bA, HEAD_DIM), lambda kh, t: (kh, t, 0)),
        out_shape=jax.ShapeDtypeStruct((KH, n, HEAD_DIM), f32),
    )(q3, k3, v3, rpe)

    out = pl.pallas_call(
        _oproj_kernel,
        grid=(n // bT, NUM_EXPERT),
        in_specs=[
            pl.BlockSpec((KH, bT, HEAD_DIM), lambda i, e: (0, i, 0)),
            pl.BlockSpec((bT, TOP_K), lambda i, e: (i, 0)),
            pl.BlockSpec((bT, TOP_K), lambda i, e: (i, 0)),
            pl.BlockSpec((1, EXPERT_DIM, D), lambda i, e: (e, 0, 0)),
        ],
        out_specs=pl.BlockSpec((bT, D), lambda i, e: (i, 0)),
        out_shape=jax.ShapeDtypeStruct((n, D), f32),
    )(y3, idx, gates, Wo)

    return out.reshape(T, B, D)


# speedup vs baseline: 4459.7150x; 1.0174x over previous
import functools

import jax
import jax.numpy as jnp
from jax.experimental import pallas as pl

EMBED_DIM = 1024
NUM_EXPERT = 16
TOP_K = 2
EXPERT_DIM = 256
HEAD_DIM = 64
NUM_HEADS = EXPERT_DIM // HEAD_DIM
MAX_POS = 64
SCALING = HEAD_DIM ** (-0.25)


def _gate_kernel(x_ref, wg_ref, idx_ref, gate_ref):
    logits = x_ref[...] @ wg_ref[...]
    e_iota = jax.lax.broadcasted_iota(jnp.int32, logits.shape, 1)
    m1 = jnp.max(logits, axis=1, keepdims=True)
    i1 = jnp.min(jnp.where(logits == m1, e_iota, NUM_EXPERT), axis=1,
                 keepdims=True)
    masked = jnp.where(e_iota == i1, -jnp.inf, logits)
    m2 = jnp.max(masked, axis=1, keepdims=True)
    i2 = jnp.min(jnp.where(masked == m2, e_iota, NUM_EXPERT), axis=1,
                 keepdims=True)
    g1 = jax.nn.sigmoid(m1 - m2)
    idx_ref[...] = jnp.concatenate([i1, i2], axis=1)
    gate_ref[...] = jnp.concatenate([g1, 1.0 - g1], axis=1)


def _qproj_kernel(x_ref, wq_ref, idx_ref, q_ref):
    e = pl.program_id(1)

    @pl.when(e == 0)
    def _():
        q_ref[...] = jnp.zeros_like(q_ref)

    p = ((x_ref[...] @ wq_ref[0]) * SCALING).astype(jnp.bfloat16)
    idx = idx_ref[...]
    for k in range(TOP_K):
        ps = jnp.where(idx[:, k:k + 1] == e, p, jnp.bfloat16(0))
        for h in range(NUM_HEADS):
            q_ref[k * NUM_HEADS + h] += ps[:, h * HEAD_DIM:(h + 1) * HEAD_DIM]


def _kv_kernel(xk_ref, xv_ref, wk_ref, wv_ref, k_ref, v_ref):
    kp = ((xk_ref[...] @ wk_ref[...]) * SCALING).astype(jnp.bfloat16)
    vp = (xv_ref[...] @ wv_ref[...]).astype(jnp.bfloat16)
    for h in range(NUM_HEADS):
        k_ref[h] = kp[:, h * HEAD_DIM:(h + 1) * HEAD_DIM]
        v_ref[h] = vp[:, h * HEAD_DIM:(h + 1) * HEAD_DIM]


def _attn_kernel(q_ref, k_ref, v_ref, e_ref, y_ref, *, block_t):
    t = pl.program_id(1)
    qs = q_ref[0]
    scores = jax.lax.dot_general(
        qs, k_ref[0], (((1,), (1,)), ((), ())),
        preferred_element_type=jnp.float32)
    rlog = jax.lax.dot_general(
        qs, e_ref[0], (((1,), (0,)), ((), ())),
        preferred_element_type=jnp.float32)
    i0 = t * block_t
    ii = jax.lax.broadcasted_iota(jnp.int32, scores.shape, 0) + i0
    jj = jax.lax.broadcasted_iota(jnp.int32, scores.shape, 1)
    d = jnp.clip(jj - ii, 1 - MAX_POS, MAX_POS - 1) + MAX_POS
    scores = scores + jnp.take_along_axis(rlog, d, axis=1)
    m = jnp.max(scores, axis=1, keepdims=True)
    p = jnp.exp(scores - m)
    l = jnp.sum(p, axis=1, keepdims=True)
    pv = jax.lax.dot_general(
        p.astype(jnp.bfloat16), v_ref[0], (((1,), (0,)), ((), ())),
        preferred_element_type=jnp.float32)
    y_ref[0] = pv / l


def _oproj_kernel(y_ref, idx_ref, gate_ref, wo_ref, o_ref):
    e = pl.program_id(1)

    @pl.when(e == 0)
    def _():
        o_ref[...] = jnp.zeros_like(o_ref)

    idx = idx_ref[...]
    g = gate_ref[...]
    z = None
    for k in range(TOP_K):
        w = jnp.where(idx[:, k:k + 1] == e, g[:, k:k + 1], 0.0)
        yk = jnp.concatenate(
            [y_ref[k * NUM_HEADS + h] for h in range(NUM_HEADS)], axis=1)
        zk = yk * w
        z = zk if z is None else z + zk
    o_ref[...] += z @ wo_ref[0]


def kernel(query, key, value, Wg, Wq, Wk, Wv, Wo, rel_pos_emb):
    T, B, D = query.shape
    S = key.shape[0]
    n = T * B
    x = query.reshape(n, D)
    xk = key.reshape(S * B, D)
    xv = value.reshape(S * B, D)
    KH = TOP_K * NUM_HEADS
    f32 = jnp.float32

    bT = 512
    idx, gates = pl.pallas_call(
        _gate_kernel,
        grid=(n // bT,),
        in_specs=[
            pl.BlockSpec((bT, D), lambda i: (i, 0)),
            pl.BlockSpec((D, NUM_EXPERT), lambda i: (0, 0)),
        ],
        out_specs=[
            pl.BlockSpec((bT, TOP_K), lambda i: (i, 0)),
            pl.BlockSpec((bT, TOP_K), lambda i: (i, 0)),
        ],
        out_shape=[
            jax.ShapeDtypeStruct((n, TOP_K), jnp.int32),
            jax.ShapeDtypeStruct((n, TOP_K), f32),
        ],
    )(x, Wg)

    q3 = pl.pallas_call(
        _qproj_kernel,
        grid=(n // bT, NUM_EXPERT),
        in_specs=[
            pl.BlockSpec((bT, D), lambda i, e: (i, 0)),
            pl.BlockSpec((1, D, EXPERT_DIM), lambda i, e: (e, 0, 0)),
            pl.BlockSpec((bT, TOP_K), lambda i, e: (i, 0)),
        ],
        out_specs=pl.BlockSpec((KH, bT, HEAD_DIM), lambda i, e: (0, i, 0)),
        out_shape=jax.ShapeDtypeStruct((KH, n, HEAD_DIM), jnp.bfloat16),
    )(x, Wq, idx)

    k3, v3 = pl.pallas_call(
        _kv_kernel,
        grid=(S * B // bT,),
        in_specs=[
            pl.BlockSpec((bT, D), lambda i: (i, 0)),
            pl.BlockSpec((bT, D), lambda i: (i, 0)),
            pl.BlockSpec((D, EXPERT_DIM), lambda i: (0, 0)),
            pl.BlockSpec((D, EXPERT_DIM), lambda i: (0, 0)),
        ],
        out_specs=[
            pl.BlockSpec((NUM_HEADS, bT, HEAD_DIM), lambda i: (0, i, 0)),
            pl.BlockSpec((NUM_HEADS, bT, HEAD_DIM), lambda i: (0, i, 0)),
        ],
        out_shape=[
            jax.ShapeDtypeStruct((NUM_HEADS, S * B, HEAD_DIM), jnp.bfloat16),
            jax.ShapeDtypeStruct((NUM_HEADS, S * B, HEAD_DIM), jnp.bfloat16),
        ],
    )(xk, xv, Wk, Wv)

    rpe = rel_pos_emb[:, :, :2 * MAX_POS].astype(jnp.bfloat16)

    bA = 256
    y3 = pl.pallas_call(
        functools.partial(_attn_kernel, block_t=bA),
        grid=(KH, T // bA),
        in_specs=[
            pl.BlockSpec((1, bA, HEAD_DIM), lambda kh, t: (kh, t, 0)),
            pl.BlockSpec((1, S, HEAD_DIM), lambda kh, t: (kh % NUM_HEADS, 0, 0)),
            pl.BlockSpec((1, S, HEAD_DIM), lambda kh, t: (kh % NUM_HEADS, 0, 0)),
            pl.BlockSpec((1, HEAD_DIM, 2 * MAX_POS),
                         lambda kh, t: (kh % NUM_HEADS, 0, 0)),
        ],
        out_specs=pl.BlockSpec((1, bA, HEAD_DIM), lambda kh, t: (kh, t, 0)),
        out_shape=jax.ShapeDtypeStruct((KH, n, HEAD_DIM), f32),
    )(q3, k3, v3, rpe)

    out = pl.pallas_call(
        _oproj_kernel,
        grid=(n // bT, NUM_EXPERT),
        in_specs=[
            pl.BlockSpec((KH, bT, HEAD_DIM), lambda i, e: (0, i, 0)),
            pl.BlockSpec((bT, TOP_K), lambda i, e: (i, 0)),
            pl.BlockSpec((bT, TOP_K), lambda i, e: (i, 0)),
            pl.BlockSpec((1, EXPERT_DIM, D), lambda i, e: (e, 0, 0)),
        ],
        out_specs=pl.BlockSpec((bT, D), lambda i, e: (i, 0)),
        out_shape=jax.ShapeDtypeStruct((n, D), f32),
    )(y3, idx, gates, Wo)

    return out.reshape(T, B, D)


# speedup vs baseline: 4763.8705x; 1.0682x over previous
import functools

import jax
import jax.numpy as jnp
from jax.experimental import pallas as pl

EMBED_DIM = 1024
NUM_EXPERT = 16
TOP_K = 2
EXPERT_DIM = 256
HEAD_DIM = 64
NUM_HEADS = EXPERT_DIM // HEAD_DIM
MAX_POS = 64
SCALING = HEAD_DIM ** (-0.25)


def _gate_kernel(x_ref, wg_ref, idx_ref, gate_ref):
    logits = x_ref[...] @ wg_ref[...]
    e_iota = jax.lax.broadcasted_iota(jnp.int32, logits.shape, 1)
    m1 = jnp.max(logits, axis=1, keepdims=True)
    i1 = jnp.min(jnp.where(logits == m1, e_iota, NUM_EXPERT), axis=1,
                 keepdims=True)
    masked = jnp.where(e_iota == i1, -jnp.inf, logits)
    m2 = jnp.max(masked, axis=1, keepdims=True)
    i2 = jnp.min(jnp.where(masked == m2, e_iota, NUM_EXPERT), axis=1,
                 keepdims=True)
    g1 = jax.nn.sigmoid(m1 - m2)
    idx_ref[...] = jnp.concatenate([i1, i2], axis=1)
    gate_ref[...] = jnp.concatenate([g1, 1.0 - g1], axis=1)


def _qproj_kernel(x_ref, wq_ref, idx_ref, q_ref):
    e = pl.program_id(1)

    @pl.when(e == 0)
    def _():
        q_ref[...] = jnp.zeros_like(q_ref)

    p = ((x_ref[...] @ wq_ref[0]) * SCALING).astype(jnp.bfloat16)
    idx = idx_ref[...]
    for k in range(TOP_K):
        sel = idx[:, k:k + 1] == e
        q_ref[:, k * EXPERT_DIM:(k + 1) * EXPERT_DIM] += jnp.where(
            sel, p, jnp.bfloat16(0))


def _kv_kernel(xk_ref, xv_ref, wk_ref, wv_ref, k_ref, v_ref):
    k_ref[...] = ((xk_ref[...] @ wk_ref[...]) * SCALING).astype(jnp.bfloat16)
    v_ref[...] = (xv_ref[...] @ wv_ref[...]).astype(jnp.bfloat16)


def _attn_kernel(q_ref, k_ref, v_ref, e_ref, y_ref, *, block_t):
    t = pl.program_id(1)
    i0 = t * block_t
    shape = (block_t, k_ref.shape[0])
    ii = jax.lax.broadcasted_iota(jnp.int32, shape, 0) + i0
    jj = jax.lax.broadcasted_iota(jnp.int32, shape, 1)
    d = jnp.clip(jj - ii, 1 - MAX_POS, MAX_POS - 1) + MAX_POS
    kk = k_ref[...]
    vv = v_ref[...]
    for h in range(NUM_HEADS):
        qh = q_ref[:, h * HEAD_DIM:(h + 1) * HEAD_DIM]
        kh = kk[:, h * HEAD_DIM:(h + 1) * HEAD_DIM]
        vh = vv[:, h * HEAD_DIM:(h + 1) * HEAD_DIM]
        scores = jax.lax.dot_general(
            qh, kh, (((1,), (1,)), ((), ())),
            preferred_element_type=jnp.float32)
        rlog = jax.lax.dot_general(
            qh, e_ref[h], (((1,), (0,)), ((), ())),
            preferred_element_type=jnp.float32)
        scores = scores + jnp.take_along_axis(rlog, d, axis=1)
        m = jnp.max(scores, axis=1, keepdims=True)
        p = jnp.exp(scores - m)
        l = jnp.sum(p, axis=1, keepdims=True)
        pv = jax.lax.dot_general(
            p.astype(jnp.bfloat16), vh, (((1,), (0,)), ((), ())),
            preferred_element_type=jnp.float32)
        y_ref[:, h * HEAD_DIM:(h + 1) * HEAD_DIM] = pv / l


def _oproj_kernel(y_ref, idx_ref, gate_ref, wo_ref, o_ref):
    e = pl.program_id(1)

    @pl.when(e == 0)
    def _():
        o_ref[...] = jnp.zeros_like(o_ref)

    idx = idx_ref[...]
    g = gate_ref[...]
    z = None
    for k in range(TOP_K):
        w = jnp.where(idx[:, k:k + 1] == e, g[:, k:k + 1], 0.0)
        zk = y_ref[:, k * EXPERT_DIM:(k + 1) * EXPERT_DIM] * w
        z = zk if z is None else z + zk
    o_ref[...] += z @ wo_ref[0]


def kernel(query, key, value, Wg, Wq, Wk, Wv, Wo, rel_pos_emb):
    T, B, D = query.shape
    S = key.shape[0]
    n = T * B
    x = query.reshape(n, D)
    xk = key.reshape(S * B, D)
    xv = value.reshape(S * B, D)
    f32 = jnp.float32
    bf16 = jnp.bfloat16

    bT = 512
    idx, gates = pl.pallas_call(
        _gate_kernel,
        grid=(n // bT,),
        in_specs=[
            pl.BlockSpec((bT, D), lambda i: (i, 0)),
            pl.BlockSpec((D, NUM_EXPERT), lambda i: (0, 0)),
        ],
        out_specs=[
            pl.BlockSpec((bT, TOP_K), lambda i: (i, 0)),
            pl.BlockSpec((bT, TOP_K), lambda i: (i, 0)),
        ],
        out_shape=[
            jax.ShapeDtypeStruct((n, TOP_K), jnp.int32),
            jax.ShapeDtypeStruct((n, TOP_K), f32),
        ],
    )(x, Wg)

    q = pl.pallas_call(
        _qproj_kernel,
        grid=(n // bT, NUM_EXPERT),
        in_specs=[
            pl.BlockSpec((bT, D), lambda i, e: (i, 0)),
            pl.BlockSpec((1, D, EXPERT_DIM), lambda i, e: (e, 0, 0)),
            pl.BlockSpec((bT, TOP_K), lambda i, e: (i, 0)),
        ],
        out_specs=pl.BlockSpec((bT, TOP_K * EXPERT_DIM), lambda i, e: (i, 0)),
        out_shape=jax.ShapeDtypeStruct((n, TOP_K * EXPERT_DIM), bf16),
    )(x, Wq, idx)

    kp, vp = pl.pallas_call(
        _kv_kernel,
        grid=(S * B // bT,),
        in_specs=[
            pl.BlockSpec((bT, D), lambda i: (i, 0)),
            pl.BlockSpec((bT, D), lambda i: (i, 0)),
            pl.BlockSpec((D, EXPERT_DIM), lambda i: (0, 0)),
            pl.BlockSpec((D, EXPERT_DIM), lambda i: (0, 0)),
        ],
        out_specs=[
            pl.BlockSpec((bT, EXPERT_DIM), lambda i: (i, 0)),
            pl.BlockSpec((bT, EXPERT_DIM), lambda i: (i, 0)),
        ],
        out_shape=[
            jax.ShapeDtypeStruct((S * B, EXPERT_DIM), bf16),
            jax.ShapeDtypeStruct((S * B, EXPERT_DIM), bf16),
        ],
    )(xk, xv, Wk, Wv)

    rpe = rel_pos_emb[:, :, :2 * MAX_POS].astype(bf16)

    bA = 256
    y = pl.pallas_call(
        functools.partial(_attn_kernel, block_t=bA),
        grid=(TOP_K, T // bA),
        in_specs=[
            pl.BlockSpec((bA, EXPERT_DIM), lambda k, t: (t, k)),
            pl.BlockSpec((S, EXPERT_DIM), lambda k, t: (0, 0)),
            pl.BlockSpec((S, EXPERT_DIM), lambda k, t: (0, 0)),
            pl.BlockSpec((NUM_HEADS, HEAD_DIM, 2 * MAX_POS),
                         lambda k, t: (0, 0, 0)),
        ],
        out_specs=pl.BlockSpec((bA, EXPERT_DIM), lambda k, t: (t, k)),
        out_shape=jax.ShapeDtypeStruct((n, TOP_K * EXPERT_DIM), f32),
    )(q, kp, vp, rpe)

    out = pl.pallas_call(
        _oproj_kernel,
        grid=(n // bT, NUM_EXPERT),
        in_specs=[
            pl.BlockSpec((bT, TOP_K * EXPERT_DIM), lambda i, e: (i, 0)),
            pl.BlockSpec((bT, TOP_K), lambda i, e: (i, 0)),
            pl.BlockSpec((bT, TOP_K), lambda i, e: (i, 0)),
            pl.BlockSpec((1, EXPERT_DIM, D), lambda i, e: (e, 0, 0)),
        ],
        out_specs=pl.BlockSpec((bT, D), lambda i, e: (i, 0)),
        out_shape=jax.ShapeDtypeStruct((n, D), f32),
    )(y, idx, gates, Wo)

    return out.reshape(T, B, D)
